# baseline (speedup 1.0000x reference)
import functools

import jax
import jax.numpy as jnp
from jax import lax
from jax.experimental import pallas as pl
from jax.experimental.pallas import tpu as pltpu
from jax.experimental.pallas import tpu_sc as plsc

B, V, N, M, IMG = 4, 8, 8192, 512, 224
NC, NS, L = 2, 16, 16
NW = NC * NS
PTS_PER_TILE = (B * N) // NW
VECS = PTS_PER_TILE // L
PAD = 544
EXT = M + 2 * PAD
SIDE = 8
BIG = 1e9


def _tile_body(pxy, pcf, maskfull, zfull, bb, invc, vidv,
               out,
               px_v, py_v, z_v, pc3_v,
               mask_v, bxs_v, bys_v, inv_v, out3_v, vid_v, sem):
    wid = lax.axis_index("c") * NS + lax.axis_index("s")
    batch = wid // (NW // B)
    chunk = wid % (NW // B)
    base = wid * PTS_PER_TILE

    pltpu.sync_copy(vidv, vid_v)
    vid = jnp.max(vid_v[...])
    view_base = (batch * V + vid) * (IMG * IMG)
    zoff = (batch * V + vid) * N + chunk * PTS_PER_TILE

    cps = [
        pltpu.async_copy(pxy.at[pl.ds(base, PTS_PER_TILE)], px_v, sem),
        pltpu.async_copy(pxy.at[pl.ds(B * N + base, PTS_PER_TILE)], py_v, sem),
        pltpu.async_copy(zfull.at[pl.ds(zoff, PTS_PER_TILE)], z_v, sem),
        pltpu.async_copy(pcf.at[pl.ds(base * 3, PTS_PER_TILE * 3)], pc3_v, sem),
        pltpu.async_copy(maskfull.at[pl.ds(view_base, IMG * IMG)], mask_v, sem),
        pltpu.async_copy(bb.at[pl.ds(batch * 2 * EXT, EXT)], bxs_v, sem),
        pltpu.async_copy(bb.at[pl.ds(batch * 2 * EXT + EXT, EXT)], bys_v, sem),
        pltpu.async_copy(invc.at[pl.ds(batch * 4 * 3 * L, 4 * 3 * L)], inv_v, sem),
    ]
    for c in cps:
        c.wait()

    lane3 = lax.iota(jnp.int32, L) * 3

    def point_vec(v, carry):
        s = v * L
        pxf = px_v[pl.ds(s, L)]
        pyf = py_v[pl.ds(s, L)]
        pxn = pxf / 224.0
        pyn = pyf / 224.0

        blo = jnp.zeros((L,), dtype=jnp.int32)
        bhi = jnp.full((L,), M, dtype=jnp.int32)
        for _ in range(9):
            mid = (blo + bhi) >> 1
            vmid = plsc.load_gather(bxs_v, [mid + PAD])
            le = vmid <= pxn
            blo = jnp.where(le, mid + 1, blo)
            bhi = jnp.where(le, bhi, mid)
        clo0 = blo + (PAD - SIDE)
        chi0 = blo + PAD

        def body(st):
            clo, chi, bd, bt, _, _ = st
            pairs = []
            for u in range(SIDE):
                t = clo + u
                bx = plsc.load_gather(bxs_v, [t])
                by = plsc.load_gather(bys_v, [t])
                dx = pxn - bx
                dy = pyn - by
                pairs.append((dx * dx + dy * dy, t))
                if u == 0:
                    glo_last = dx
            for u in range(SIDE):
                t = chi + u
                bx = plsc.load_gather(bxs_v, [t])
                by = plsc.load_gather(bys_v, [t])
                dx = pxn - bx
                dy = pyn - by
                pairs.append((dx * dx + dy * dy, t))
                if u == SIDE - 1:
                    ghi_last = -dx
            while len(pairs) > 1:
                nxt = []
                for q in range(0, len(pairs), 2):
                    a, b = pairs[q], pairs[q + 1]
                    m = b[0] < a[0]
                    nxt.append((jnp.where(m, b[0], a[0]),
                                jnp.where(m, b[1], a[1])))
                pairs = nxt
            dg, tg = pairs[0]
            m = dg < bd
            bd = jnp.where(m, dg, bd)
            bt = jnp.where(m, tg, bt)
            return (clo - SIDE, chi + SIDE, bd, bt, glo_last, ghi_last)

        def cond(st):
            _, _, bd, _, glo_last, ghi_last = st
            gmin = jnp.minimum(glo_last, ghi_last)
            alive = jnp.where(gmin * gmin <= bd, 1, 0).astype(jnp.int32)
            return jnp.max(alive) > 0

        bd0 = jnp.full((L,), jnp.inf, dtype=jnp.float32)
        zf32 = jnp.zeros((L,), dtype=jnp.float32)
        st = (clo0, chi0, bd0, chi0, zf32, zf32)
        st = body(st)
        st = lax.while_loop(cond, body, st)
        bt = st[3]

        nbx = plsc.load_gather(bxs_v, [bt])
        nby = plsc.load_gather(bys_v, [bt])

        pxi = pxf.astype(jnp.int32)
        pyi = pyf.astype(jnp.int32)
        pyc = jnp.clip(pyi, 0, IMG - 1)
        pxc = jnp.clip(pxi, 0, IMG - 1)
        mval = plsc.load_gather(mask_v, [pyc * IMG + pxc])
        inimg = (pyi == pyc) & (pxi == pxc)
        flag = (mval == 0.0) | jnp.logical_not(inimg)

        zv = z_v[pl.ds(s, L)]
        b0 = (nbx * 224.0) * zv
        b1 = (nby * 224.0) * zv
        s3 = s * 3
        for cix in range(3):
            a0 = inv_v[pl.ds((0 * 3 + cix) * L, L)]
            a1 = inv_v[pl.ds((1 * 3 + cix) * L, L)]
            a2 = inv_v[pl.ds((2 * 3 + cix) * L, L)]
            a3 = inv_v[pl.ds((3 * 3 + cix) * L, L)]
            bc = b0 * a0 + b1 * a1 + zv * a2 + a3
            idx = lane3 + (s3 + cix)
            pcv = plsc.load_gather(pc3_v, [idx])
            plsc.store_scatter(out3_v, [idx], jnp.where(flag, bc, pcv))
        return carry

    lax.fori_loop(0, VECS, point_vec, 0)

    pltpu.sync_copy(out3_v, out.at[pl.ds(base * 3, PTS_PER_TILE * 3)])


@functools.partial(
    pl.kernel,
    out_type=jax.ShapeDtypeStruct((B * N * 3,), jnp.float32),
    mesh=plsc.VectorSubcoreMesh(core_axis_name="c", subcore_axis_name="s"),
    compiler_params=pltpu.CompilerParams(needs_layout_passes=False),
    scratch_types=[
        pltpu.VMEM((PTS_PER_TILE,), jnp.float32),
        pltpu.VMEM((PTS_PER_TILE,), jnp.float32),
        pltpu.VMEM((PTS_PER_TILE,), jnp.float32),
        pltpu.VMEM((PTS_PER_TILE * 3,), jnp.float32),
        pltpu.VMEM((IMG * IMG,), jnp.float32),
        pltpu.VMEM((EXT,), jnp.float32),
        pltpu.VMEM((EXT,), jnp.float32),
        pltpu.VMEM((4 * 3 * L,), jnp.float32),
        pltpu.VMEM((PTS_PER_TILE * 3,), jnp.float32),
        pltpu.VMEM((L,), jnp.int32),
        pltpu.SemaphoreType.DMA,
    ],
)
def _sc_calibrate(*refs):
    _tile_body(*refs)


def kernel(pc, mask, bounds, view_id, inv_param, proj_fine, proj_finez):
    projv = lax.dynamic_index_in_dim(proj_fine, view_id, axis=1, keepdims=False)
    boundsv = lax.dynamic_index_in_dim(bounds, view_id, axis=1, keepdims=False)
    invv = lax.dynamic_index_in_dim(inv_param, view_id, axis=1, keepdims=False)

    pxy = jnp.concatenate([jnp.round(projv[..., 0]).reshape(B * N),
                           jnp.round(224.0 - projv[..., 1]).reshape(B * N)])

    bn = boundsv / 224.0
    bxsrt, bysrt = lax.sort((bn[..., 0], bn[..., 1]), dimension=1, num_keys=1)
    bxsp = jnp.pad(bxsrt, ((0, 0), (PAD, PAD)),
                   constant_values=((0.0, 0.0), (-BIG, BIG)))
    bysp = jnp.pad(bysrt, ((0, 0), (PAD, PAD)))
    bb = jnp.concatenate([bxsp[:, None, :], bysp[:, None, :]],
                         axis=1).reshape(B * 2 * EXT)

    invc = jnp.broadcast_to(invv[:, :, :3, None], (B, 4, 3, L)).reshape(B * 4 * 3 * L)
    vidv = jnp.full((L,), view_id, dtype=jnp.int32)

    out = _sc_calibrate(pxy, pc.reshape(B * N * 3), mask.reshape(B * V * IMG * IMG),
                        proj_finez.reshape(B * V * N), bb, invc, vidv)
    return out.reshape(B, N, 3)


# speedup vs baseline: 1.9338x; 1.9338x over previous
import functools

import jax
import jax.numpy as jnp
from jax import lax
from jax.experimental import pallas as pl
from jax.experimental.pallas import tpu as pltpu
from jax.experimental.pallas import tpu_sc as plsc

B, V, N, M, IMG = 4, 8, 8192, 512, 224
NC, NS, L = 2, 16, 16
NW = NC * NS
PTS_PER_TILE = (B * N) // NW
VECS = PTS_PER_TILE // L
PAD = 544
EXT = M + 2 * PAD
SIDE = 12
TBLW = 232
BIG = 1e9


def _tile_body(pts, maskf, bb, invc, post,
               out,
               px_v, py_v, z_v, pcx_v, pcy_v, pcz_v,
               mask_v, bxs_v, bys_v, inv_v, post_v, ox_v, oy_v, oz_v, sem):
    wid = lax.axis_index("c") * NS + lax.axis_index("s")
    batch = wid // (NW // B)
    base = wid * PTS_PER_TILE

    cps = [
        pltpu.async_copy(pts.at[pl.ds(0 * B * N + base, PTS_PER_TILE)], px_v, sem),
        pltpu.async_copy(pts.at[pl.ds(1 * B * N + base, PTS_PER_TILE)], py_v, sem),
        pltpu.async_copy(pts.at[pl.ds(2 * B * N + base, PTS_PER_TILE)], z_v, sem),
        pltpu.async_copy(pts.at[pl.ds(3 * B * N + base, PTS_PER_TILE)], pcx_v, sem),
        pltpu.async_copy(pts.at[pl.ds(4 * B * N + base, PTS_PER_TILE)], pcy_v, sem),
        pltpu.async_copy(pts.at[pl.ds(5 * B * N + base, PTS_PER_TILE)], pcz_v, sem),
        pltpu.async_copy(maskf.at[pl.ds(batch * IMG * IMG, IMG * IMG)], mask_v, sem),
        pltpu.async_copy(bb.at[pl.ds(batch * 2 * EXT, EXT)], bxs_v, sem),
        pltpu.async_copy(bb.at[pl.ds(batch * 2 * EXT + EXT, EXT)], bys_v, sem),
        pltpu.async_copy(invc.at[pl.ds(batch * 4 * 3 * L, 4 * 3 * L)], inv_v, sem),
        pltpu.async_copy(post.at[pl.ds(batch * TBLW, TBLW)], post_v, sem),
    ]
    for c in cps:
        c.wait()

    def point_vec(v, carry):
        s = v * L
        pxf = px_v[pl.ds(s, L)]
        pyf = py_v[pl.ds(s, L)]
        pxn = pxf / 224.0
        pyn = pyf / 224.0

        pxi = pxf.astype(jnp.int32)
        pyi = pyf.astype(jnp.int32)
        pyc = jnp.clip(pyi, 0, IMG - 1)
        pxc = jnp.clip(pxi, 0, IMG - 1)

        pos = plsc.load_gather(post_v, [pxc])
        clo0 = pos + (PAD - SIDE)
        chi0 = pos + PAD

        def body(st):
            clo, chi, bd, bt, _, _ = st
            pairs = []
            for u in range(SIDE):
                t = clo + u
                bx = plsc.load_gather(bxs_v, [t])
                by = plsc.load_gather(bys_v, [t])
                dx = pxn - bx
                dy = pyn - by
                pairs.append((dx * dx + dy * dy, t))
                if u == 0:
                    glo_last = dx
            for u in range(SIDE):
                t = chi + u
                bx = plsc.load_gather(bxs_v, [t])
                by = plsc.load_gather(bys_v, [t])
                dx = pxn - bx
                dy = pyn - by
                pairs.append((dx * dx + dy * dy, t))
                if u == SIDE - 1:
                    ghi_last = -dx
            while len(pairs) > 1:
                nxt = []
                for q in range(0, len(pairs) - 1, 2):
                    a, b = pairs[q], pairs[q + 1]
                    m = b[0] < a[0]
                    nxt.append((jnp.where(m, b[0], a[0]),
                                jnp.where(m, b[1], a[1])))
                if len(pairs) % 2:
                    nxt.append(pairs[-1])
                pairs = nxt
            dg, tg = pairs[0]
            m = dg < bd
            bd = jnp.where(m, dg, bd)
            bt = jnp.where(m, tg, bt)
            return (clo - SIDE, chi + SIDE, bd, bt, glo_last, ghi_last)

        def cond(st):
            _, _, bd, _, glo_last, ghi_last = st
            gmin = jnp.minimum(glo_last, ghi_last)
            alive = jnp.where(gmin * gmin <= bd, 1, 0).astype(jnp.int32)
            return jnp.max(alive) > 0

        bd0 = jnp.full((L,), jnp.inf, dtype=jnp.float32)
        zf32 = jnp.zeros((L,), dtype=jnp.float32)
        st = (clo0, chi0, bd0, chi0, zf32, zf32)
        st = body(st)
        st = lax.while_loop(cond, body, st)
        bt = st[3]

        nbx = plsc.load_gather(bxs_v, [bt])
        nby = plsc.load_gather(bys_v, [bt])

        mval = plsc.load_gather(mask_v, [pyc * IMG + pxc])
        inimg = (pyi == pyc) & (pxi == pxc)
        flag = (mval == 0.0) | jnp.logical_not(inimg)

        zv = z_v[pl.ds(s, L)]
        b0 = (nbx * 224.0) * zv
        b1 = (nby * 224.0) * zv
        pc_vs = (pcx_v, pcy_v, pcz_v)
        o_vs = (ox_v, oy_v, oz_v)
        for cix in range(3):
            a0 = inv_v[pl.ds((0 * 3 + cix) * L, L)]
            a1 = inv_v[pl.ds((1 * 3 + cix) * L, L)]
            a2 = inv_v[pl.ds((2 * 3 + cix) * L, L)]
            a3 = inv_v[pl.ds((3 * 3 + cix) * L, L)]
            bc = b0 * a0 + b1 * a1 + zv * a2 + a3
            o_vs[cix][pl.ds(s, L)] = jnp.where(flag, bc, pc_vs[cix][pl.ds(s, L)])
        return carry

    lax.fori_loop(0, VECS, point_vec, 0)

    pltpu.sync_copy(ox_v, out.at[pl.ds(0 * B * N + base, PTS_PER_TILE)])
    pltpu.sync_copy(oy_v, out.at[pl.ds(1 * B * N + base, PTS_PER_TILE)])
    pltpu.sync_copy(oz_v, out.at[pl.ds(2 * B * N + base, PTS_PER_TILE)])


@functools.partial(
    pl.kernel,
    out_type=jax.ShapeDtypeStruct((3 * B * N,), jnp.float32),
    mesh=plsc.VectorSubcoreMesh(core_axis_name="c", subcore_axis_name="s"),
    compiler_params=pltpu.CompilerParams(needs_layout_passes=False),
    scratch_types=[
        pltpu.VMEM((PTS_PER_TILE,), jnp.float32),
        pltpu.VMEM((PTS_PER_TILE,), jnp.float32),
        pltpu.VMEM((PTS_PER_TILE,), jnp.float32),
        pltpu.VMEM((PTS_PER_TILE,), jnp.float32),
        pltpu.VMEM((PTS_PER_TILE,), jnp.float32),
        pltpu.VMEM((PTS_PER_TILE,), jnp.float32),
        pltpu.VMEM((IMG * IMG,), jnp.float32),
        pltpu.VMEM((EXT,), jnp.float32),
        pltpu.VMEM((EXT,), jnp.float32),
        pltpu.VMEM((4 * 3 * L,), jnp.float32),
        pltpu.VMEM((TBLW,), jnp.int32),
        pltpu.VMEM((PTS_PER_TILE,), jnp.float32),
        pltpu.VMEM((PTS_PER_TILE,), jnp.float32),
        pltpu.VMEM((PTS_PER_TILE,), jnp.float32),
        pltpu.SemaphoreType.DMA,
    ],
)
def _sc_calibrate(*refs):
    _tile_body(*refs)


def kernel(pc, mask, bounds, view_id, inv_param, proj_fine, proj_finez):
    projv = lax.dynamic_index_in_dim(proj_fine, view_id, axis=1, keepdims=False)
    maskv = lax.dynamic_index_in_dim(mask, view_id, axis=1, keepdims=False)
    boundsv = lax.dynamic_index_in_dim(bounds, view_id, axis=1, keepdims=False)
    invv = lax.dynamic_index_in_dim(inv_param, view_id, axis=1, keepdims=False)
    zv = lax.dynamic_index_in_dim(proj_finez, view_id, axis=1, keepdims=False)

    pxr = jnp.round(projv[..., 0]).reshape(B * N)
    pyr = jnp.round(224.0 - projv[..., 1]).reshape(B * N)
    pts = jnp.concatenate([pxr, pyr, zv.reshape(B * N),
                           pc[..., 0].reshape(B * N),
                           pc[..., 1].reshape(B * N),
                           pc[..., 2].reshape(B * N)])

    maskf = maskv.reshape(B * IMG * IMG)

    bn = boundsv / 224.0
    bxn = bn[..., 0]
    byn = bn[..., 1]
    bxsrt, bysrt = lax.sort((bxn, byn), dimension=1, num_keys=1)
    bxsp = jnp.pad(bxsrt, ((0, 0), (PAD, PAD)),
                   constant_values=((0.0, 0.0), (-BIG, BIG)))
    bysp = jnp.pad(bysrt, ((0, 0), (PAD, PAD)))
    bb = jnp.concatenate([bxsp[:, None, :], bysp[:, None, :]],
                         axis=1).reshape(B * 2 * EXT)

    invc = jnp.broadcast_to(invv[:, :, :3, None], (B, 4, 3, L)).reshape(B * 4 * 3 * L)

    grid = jnp.arange(226, dtype=jnp.float32) / 224.0
    post = jnp.sum((bxsrt[:, :, None] <= grid[None, None, :]).astype(jnp.int32),
                   axis=1)
    post = jnp.pad(post, ((0, 0), (0, TBLW - 226))).reshape(B * TBLW)

    out = _sc_calibrate(pts, maskf, bb, invc, post)
    return out.reshape(3, B * N).T.reshape(B, N, 3)
